# baseline (speedup 1.0000x reference)
import functools

import jax
import jax.numpy as jnp
from jax import lax
from jax.experimental import pallas as pl
from jax.experimental.pallas import tpu as pltpu
from jax.experimental.pallas import tpu_sc as plsc

EPS_ = 0.01
NGROUPS = 32
NC = 2
NS = 16
NW = NC * NS
LANES = 16

ROWS = 200
COLS = 16384
COLS_W = COLS // NW
RCHUNK = 40
NCHUNK = ROWS // RCHUNK
NVROW = COLS_W // LANES
TABLE = 48


def _body(x_hbm, g_hbm, mins_hbm, maxs_hbm, out_hbm,
          xv, gv, ov, tab_m, tab_r, ptab, sems):
    wid = lax.axis_index("s") * NC + lax.axis_index("c")
    col0 = wid * COLS_W

    pltpu.sync_copy(mins_hbm, tab_m)
    pltpu.sync_copy(maxs_hbm, tab_r)
    for j in range(NGROUPS // LANES):
        sl = pl.ds(j * LANES, LANES)
        m = tab_m[sl]
        r = tab_r[sl] - m
        s = (1.0 - 2.0 * EPS_) / r
        o = EPS_ - s * m
        s_hi = (plsc.bitcast(s, jnp.int32) + 0x8000) & jnp.int32(-65536)
        o_hi = plsc.bitcast(o, jnp.int32) + 0x8000
        o_lo = lax.shift_right_logical(o_hi, 16)
        w = s_hi | o_lo
        idx = lax.iota(jnp.int32, LANES) + (1 + j * LANES)
        plsc.store_scatter(ptab, [idx], w)

    GROUP = 8
    NBUF = 2

    def compute(b):
        @pl.loop(0, RCHUNK)
        def _(r):
            for c0 in range(0, NVROW, GROUP):
                sls = [pl.ds((c0 + k) * LANES, LANES) for k in range(GROUP)]
                gs = [gv[b, r, sl] for sl in sls]
                ws = [plsc.load_gather(ptab, [g]) for g in gs]
                xs = [xv[b, r, sl] for sl in sls]
                for k in range(GROUP):
                    s = plsc.bitcast(ws[k] & jnp.int32(-65536), jnp.float32)
                    o = plsc.bitcast(lax.shift_left(ws[k], 16), jnp.float32)
                    ov[b, r, sls[k]] = xs[k] * s + o

    def start_in(c, b):
        rows = pl.ds(c * RCHUNK, RCHUNK)
        cols = pl.ds(col0, COLS_W)
        pltpu.async_copy(x_hbm.at[rows, cols], xv.at[b], sems.at[b, 0])
        pltpu.async_copy(g_hbm.at[rows, cols], gv.at[b], sems.at[b, 1])

    def wait_in(b):
        pltpu.make_async_copy(x_hbm.at[pl.ds(0, RCHUNK), pl.ds(0, COLS_W)],
                              xv.at[b], sems.at[b, 0]).wait()
        pltpu.make_async_copy(g_hbm.at[pl.ds(0, RCHUNK), pl.ds(0, COLS_W)],
                              gv.at[b], sems.at[b, 1]).wait()

    def start_out(c, b):
        rows = pl.ds(c * RCHUNK, RCHUNK)
        cols = pl.ds(col0, COLS_W)
        pltpu.async_copy(ov.at[b], out_hbm.at[rows, cols], sems.at[b, 2])

    def wait_out(b):
        pltpu.make_async_copy(ov.at[b],
                              out_hbm.at[pl.ds(0, RCHUNK), pl.ds(0, COLS_W)],
                              sems.at[b, 2]).wait()

    start_in(0, 0)

    @pl.loop(0, NCHUNK)
    def _(c):
        b = lax.rem(c, NBUF)

        @pl.when(c + 1 < NCHUNK)
        def _():
            start_in(c + 1, lax.rem(c + 1, NBUF))

        wait_in(b)

        @pl.when(c >= NBUF)
        def _():
            wait_out(b)

        compute(b)
        start_out(c, b)

    for k in range(NBUF):
        wait_out(lax.rem(NCHUNK - NBUF + k, NBUF))


@jax.jit
def _run(xt, gt, mins, maxs):
    mesh = plsc.VectorSubcoreMesh(core_axis_name="c", subcore_axis_name="s")
    f = pl.kernel(
        _body,
        out_type=jax.ShapeDtypeStruct((ROWS, COLS), jnp.float32),
        mesh=mesh,
        compiler_params=pltpu.CompilerParams(
            needs_layout_passes=False,
            use_tc_tiling_on_sc=True,
        ),
        scratch_types=[
            pltpu.VMEM((2, RCHUNK, COLS_W), jnp.float32),
            pltpu.VMEM((2, RCHUNK, COLS_W), jnp.int32),
            pltpu.VMEM((2, RCHUNK, COLS_W), jnp.float32),
            pltpu.VMEM((NGROUPS,), jnp.float32),
            pltpu.VMEM((NGROUPS,), jnp.float32),
            pltpu.VMEM((TABLE,), jnp.int32),
            pltpu.SemaphoreType.DMA((2, 3)),
        ],
    )
    return f(xt, gt, mins, maxs)


def kernel(x, group, mins, maxs):
    return _run(x.T, group.T, mins, maxs).T


# speedup vs baseline: 1.0173x; 1.0173x over previous
import functools

import jax
import jax.numpy as jnp
from jax import lax
from jax.experimental import pallas as pl
from jax.experimental.pallas import tpu as pltpu
from jax.experimental.pallas import tpu_sc as plsc

EPS_ = 0.01
NGROUPS = 32
NC = 2
NS = 16
NW = NC * NS
LANES = 16

ROWS = 200
COLS = 16384
COLS_W = COLS // NW
RCHUNK = 8
NCHUNK = ROWS // RCHUNK
NVROW = COLS_W // LANES
TABLE = 48


def _body(x_hbm, g_hbm, mins_hbm, maxs_hbm, out_hbm,
          xv, gv, ov, tab_m, tab_r, ptab, sems):
    wid = lax.axis_index("s") * NC + lax.axis_index("c")
    col0 = wid * COLS_W

    pltpu.sync_copy(mins_hbm, tab_m)
    pltpu.sync_copy(maxs_hbm, tab_r)
    for j in range(NGROUPS // LANES):
        sl = pl.ds(j * LANES, LANES)
        m = tab_m[sl]
        r = tab_r[sl] - m
        s = (1.0 - 2.0 * EPS_) / r
        o = EPS_ - s * m
        s_hi = (plsc.bitcast(s, jnp.int32) + 0x8000) & jnp.int32(-65536)
        o_hi = plsc.bitcast(o, jnp.int32) + 0x8000
        o_lo = lax.shift_right_logical(o_hi, 16)
        w = s_hi | o_lo
        idx = lax.iota(jnp.int32, LANES) + (1 + j * LANES)
        plsc.store_scatter(ptab, [idx], w)

    GROUP = 8
    NBUF = 4

    def compute(b):
        @pl.loop(0, RCHUNK)
        def _(r):
            for c0 in range(0, NVROW, GROUP):
                sls = [pl.ds((c0 + k) * LANES, LANES) for k in range(GROUP)]
                gs = [gv[b, r, sl] for sl in sls]
                ws = [plsc.load_gather(ptab, [g]) for g in gs]
                xs = [xv[b, r, sl] for sl in sls]
                for k in range(GROUP):
                    s = plsc.bitcast(ws[k] & jnp.int32(-65536), jnp.float32)
                    o = plsc.bitcast(lax.shift_left(ws[k], 16), jnp.float32)
                    ov[b, r, sls[k]] = xs[k] * s + o

    def start_in(c, b):
        rows = pl.ds(c * RCHUNK, RCHUNK)
        cols = pl.ds(col0, COLS_W)
        pltpu.async_copy(x_hbm.at[rows, cols], xv.at[b], sems.at[b, 0])
        pltpu.async_copy(g_hbm.at[rows, cols], gv.at[b], sems.at[b, 1])

    def wait_in(b):
        pltpu.make_async_copy(x_hbm.at[pl.ds(0, RCHUNK), pl.ds(0, COLS_W)],
                              xv.at[b], sems.at[b, 0]).wait()
        pltpu.make_async_copy(g_hbm.at[pl.ds(0, RCHUNK), pl.ds(0, COLS_W)],
                              gv.at[b], sems.at[b, 1]).wait()

    def start_out(c, b):
        rows = pl.ds(c * RCHUNK, RCHUNK)
        cols = pl.ds(col0, COLS_W)
        pltpu.async_copy(ov.at[b], out_hbm.at[rows, cols], sems.at[b, 2])

    def wait_out(b):
        pltpu.make_async_copy(ov.at[b],
                              out_hbm.at[pl.ds(0, RCHUNK), pl.ds(0, COLS_W)],
                              sems.at[b, 2]).wait()

    start_in(0, 0)
    start_in(1, 1)
    start_in(2, 2)

    @pl.loop(0, NCHUNK)
    def _(c):
        b = lax.rem(c, NBUF)

        @pl.when(c + 3 < NCHUNK)
        def _():
            start_in(c + 3, lax.rem(c + 3, NBUF))

        wait_in(b)

        @pl.when(c >= NBUF)
        def _():
            wait_out(b)

        compute(b)
        start_out(c, b)

    for k in range(NBUF):
        wait_out(lax.rem(NCHUNK - NBUF + k, NBUF))


@jax.jit
def _run(xt, gt, mins, maxs):
    mesh = plsc.VectorSubcoreMesh(core_axis_name="c", subcore_axis_name="s")
    f = pl.kernel(
        _body,
        out_type=jax.ShapeDtypeStruct((ROWS, COLS), jnp.float32),
        mesh=mesh,
        compiler_params=pltpu.CompilerParams(
            needs_layout_passes=False,
            use_tc_tiling_on_sc=True,
        ),
        scratch_types=[
            pltpu.VMEM((4, RCHUNK, COLS_W), jnp.float32),
            pltpu.VMEM((4, RCHUNK, COLS_W), jnp.int32),
            pltpu.VMEM((4, RCHUNK, COLS_W), jnp.float32),
            pltpu.VMEM((NGROUPS,), jnp.float32),
            pltpu.VMEM((NGROUPS,), jnp.float32),
            pltpu.VMEM((TABLE,), jnp.int32),
            pltpu.SemaphoreType.DMA((4, 3)),
        ],
    )
    return f(xt, gt, mins, maxs)


def kernel(x, group, mins, maxs):
    return _run(x.T, group.T, mins, maxs).T
